# Initial kernel scaffold; baseline (speedup 1.0000x reference)
import jax
import jax.numpy as jnp
from jax.experimental import pallas as pl


def kernel(x, adj, node_nums, W0, al0, ar0, b0, W1, al1, ar1, b1, W2, al2, ar2, b2):
    raise NotImplementedError("write your pallas kernel here")


# speedup vs baseline: 2601.7743x; 2601.7743x over previous
import functools

import jax
import jax.numpy as jnp
from jax import lax
from jax.experimental import pallas as pl
from jax.experimental.pallas import tpu as pltpu

_H = 4


def _attention_layer(h, W_ref, al_ref, ar_ref, b_ref, mask, neg, Fo, act, mean_heads):
    feat = jnp.dot(h, W_ref[...], preferred_element_type=jnp.float32)
    outs = None
    for hd in range(_H):
        f_h = feat[:, hd * Fo:(hd + 1) * Fo]
        al_h = al_ref[hd:hd + 1, :]
        ar_h = ar_ref[hd:hd + 1, :]
        el = jnp.sum(f_h * al_h, axis=1, keepdims=True)
        er = lax.dot_general(ar_h, f_h, (((1,), (1,)), ((), ())),
                             preferred_element_type=jnp.float32)
        e = el + er
        e = jnp.where(e > 0, e, 0.2 * e)
        e = jnp.where(mask, e, neg)
        emax = jnp.max(e, axis=0, keepdims=True)
        ee = jnp.where(mask, jnp.exp(e - emax), 0.0)
        denom = jnp.sum(ee, axis=0, keepdims=True)
        alpha = ee / jnp.maximum(denom, 1e-9)
        o_h = lax.dot_general(alpha, f_h, (((0,), (0,)), ((), ())),
                              preferred_element_type=jnp.float32)
        o_h = o_h + b_ref[:, hd * Fo:(hd + 1) * Fo]
        if mean_heads:
            outs = o_h if outs is None else outs + o_h
        else:
            outs = o_h if outs is None else jnp.concatenate([outs, o_h], axis=1)
    if mean_heads:
        outs = outs * (1.0 / _H)
    if act:
        outs = jnp.maximum(outs, 0.0)
    return outs


def _gat_kernel(node_nums_ref, x_ref, adj_ref,
                W0_ref, al0_ref, ar0_ref, b0_ref,
                W1_ref, al1_ref, ar1_ref, b1_ref,
                W2_ref, al2_ref, ar2_ref, b2_ref,
                out_ref, *, Nmax, Fh, Fout):
    b = pl.program_id(0)
    nn = jnp.maximum(node_nums_ref[b], 1)
    ii = lax.broadcasted_iota(jnp.int32, (Nmax, Nmax), 0)
    jj = lax.broadcasted_iota(jnp.int32, (Nmax, Nmax), 1)
    mask = (adj_ref[0, 0] != 0) & (ii < nn) & (jj < nn)
    neg = jnp.float32(-1e30)

    h = x_ref[0, 0]
    h = _attention_layer(h, W0_ref, al0_ref, ar0_ref, b0_ref, mask, neg,
                         Fh, act=True, mean_heads=False)
    h = _attention_layer(h, W1_ref, al1_ref, ar1_ref, b1_ref, mask, neg,
                         Fh, act=True, mean_heads=False)
    h = _attention_layer(h, W2_ref, al2_ref, ar2_ref, b2_ref, mask, neg,
                         Fout, act=False, mean_heads=True)
    valid_col = lax.broadcasted_iota(jnp.int32, (Nmax, 1), 0) < nn
    out_ref[0] = jnp.where(valid_col, h, 0.0)


def kernel(x, adj, node_nums, W0, al0, ar0, b0, W1, al1, ar1, b1,
           W2, al2, ar2, b2):
    B, C, Nmax, Fin = x.shape
    Hh, Fh = al0.shape
    Fout = al2.shape[1]
    HF = Hh * Fh

    b0r = b0.reshape(1, HF)
    b1r = b1.reshape(1, HF)
    b2r = b2.reshape(1, Hh * Fout)

    def full(shape):
        return pl.BlockSpec(shape, lambda b, *_: (0,) * len(shape))

    grid_spec = pltpu.PrefetchScalarGridSpec(
        num_scalar_prefetch=1,
        grid=(B,),
        in_specs=[
            pl.BlockSpec((1, 1, Nmax, Fin), lambda b, *_: (b, 0, 0, 0)),
            pl.BlockSpec((1, 1, Nmax, Nmax), lambda b, *_: (b, 0, 0, 0)),
            full((Fin, HF)), full((Hh, Fh)), full((Hh, Fh)), full((1, HF)),
            full((HF, HF)), full((Hh, Fh)), full((Hh, Fh)), full((1, HF)),
            full((HF, Hh * Fout)), full((Hh, Fout)), full((Hh, Fout)),
            full((1, Hh * Fout)),
        ],
        out_specs=pl.BlockSpec((1, Nmax, Fout), lambda b, *_: (b, 0, 0)),
    )

    out = pl.pallas_call(
        functools.partial(_gat_kernel, Nmax=Nmax, Fh=Fh, Fout=Fout),
        grid_spec=grid_spec,
        out_shape=jax.ShapeDtypeStruct((B, Nmax, Fout), jnp.float32),
    )(node_nums.astype(jnp.int32), x, adj,
      W0, al0, ar0, b0r, W1, al1, ar1, b1r, W2, al2, ar2, b2r)
    return out


# speedup vs baseline: 2611.0927x; 1.0036x over previous
import functools

import jax
import jax.numpy as jnp
from jax import lax
from jax.experimental import pallas as pl
from jax.experimental.pallas import tpu as pltpu

_H = 4


def _attention_layer(h, W_ref, al_ref, ar_ref, b_ref, mask, neg, Fo, act, mean_heads):
    feat = jnp.dot(h, W_ref[...], preferred_element_type=jnp.float32)
    outs = None
    for hd in range(_H):
        f_h = feat[:, hd * Fo:(hd + 1) * Fo]
        al_h = al_ref[hd:hd + 1, :]
        ar_h = ar_ref[hd:hd + 1, :]
        el = jnp.sum(f_h * al_h, axis=1, keepdims=True)
        er = lax.dot_general(ar_h, f_h, (((1,), (1,)), ((), ())),
                             preferred_element_type=jnp.float32)
        e = el + er
        e = jnp.where(e > 0, e, 0.2 * e)
        e = jnp.where(mask, e, neg)
        emax = jnp.max(e, axis=0, keepdims=True)
        ee = jnp.where(mask, jnp.exp(e - emax), 0.0)
        denom = jnp.sum(ee, axis=0, keepdims=True)
        alpha = ee / jnp.maximum(denom, 1e-9)
        o_h = lax.dot_general(alpha, f_h, (((0,), (0,)), ((), ())),
                              preferred_element_type=jnp.float32)
        o_h = o_h + b_ref[:, hd * Fo:(hd + 1) * Fo]
        if mean_heads:
            outs = o_h if outs is None else outs + o_h
        else:
            outs = o_h if outs is None else jnp.concatenate([outs, o_h], axis=1)
    if mean_heads:
        outs = outs * (1.0 / _H)
    if act:
        outs = jnp.maximum(outs, 0.0)
    return outs


def _gat_kernel(node_nums_ref, x_ref, adj_ref,
                W0_ref, al0_ref, ar0_ref, b0_ref,
                W1_ref, al1_ref, ar1_ref, b1_ref,
                W2_ref, al2_ref, ar2_ref, b2_ref,
                out_ref, *, Nmax, Fh, Fout):
    b = pl.program_id(0)
    nn = jnp.maximum(node_nums_ref[b], 1)
    ii = lax.broadcasted_iota(jnp.int32, (Nmax, Nmax), 0)
    jj = lax.broadcasted_iota(jnp.int32, (Nmax, Nmax), 1)
    mask = (adj_ref[0, 0] != 0) & (ii < nn) & (jj < nn)
    neg = jnp.float32(-1e30)

    h = x_ref[0, 0]
    h = _attention_layer(h, W0_ref, al0_ref, ar0_ref, b0_ref, mask, neg,
                         Fh, act=True, mean_heads=False)
    h = _attention_layer(h, W1_ref, al1_ref, ar1_ref, b1_ref, mask, neg,
                         Fh, act=True, mean_heads=False)
    h = _attention_layer(h, W2_ref, al2_ref, ar2_ref, b2_ref, mask, neg,
                         Fout, act=False, mean_heads=True)
    valid_col = lax.broadcasted_iota(jnp.int32, (Nmax, 1), 0) < nn
    out_ref[0] = jnp.where(valid_col, h, 0.0)


def kernel(x, adj, node_nums, W0, al0, ar0, b0, W1, al1, ar1, b1,
           W2, al2, ar2, b2):
    B, C, Nmax, Fin = x.shape
    Hh, Fh = al0.shape
    Fout = al2.shape[1]
    HF = Hh * Fh

    b0r = b0.reshape(1, HF)
    b1r = b1.reshape(1, HF)
    b2r = b2.reshape(1, Hh * Fout)

    def full(shape):
        return pl.BlockSpec(shape, lambda b, *_: (0,) * len(shape))

    grid_spec = pltpu.PrefetchScalarGridSpec(
        num_scalar_prefetch=1,
        grid=(B,),
        in_specs=[
            pl.BlockSpec((1, 1, Nmax, Fin), lambda b, *_: (b, 0, 0, 0)),
            pl.BlockSpec((1, 1, Nmax, Nmax), lambda b, *_: (b, 0, 0, 0)),
            full((Fin, HF)), full((Hh, Fh)), full((Hh, Fh)), full((1, HF)),
            full((HF, HF)), full((Hh, Fh)), full((Hh, Fh)), full((1, HF)),
            full((HF, Hh * Fout)), full((Hh, Fout)), full((Hh, Fout)),
            full((1, Hh * Fout)),
        ],
        out_specs=pl.BlockSpec((1, Nmax, Fout), lambda b, *_: (b, 0, 0)),
    )

    out = pl.pallas_call(
        functools.partial(_gat_kernel, Nmax=Nmax, Fh=Fh, Fout=Fout),
        grid_spec=grid_spec,
        out_shape=jax.ShapeDtypeStruct((B, Nmax, Fout), jnp.float32),
        compiler_params=pltpu.CompilerParams(
            dimension_semantics=("parallel",)),
    )(node_nums.astype(jnp.int32), x, adj,
      W0, al0, ar0, b0r, W1, al1, ar1, b1r, W2, al2, ar2, b2r)
    return out
